# Initial kernel scaffold; baseline (speedup 1.0000x reference)
import jax
import jax.numpy as jnp
from jax.experimental import pallas as pl


def kernel(x, edge_index, edge_attr, u, batch):
    raise NotImplementedError("write your pallas kernel here")


# speedup vs baseline: 3.2243x; 3.2243x over previous
import functools

import jax
import jax.numpy as jnp
from jax import lax
from jax.experimental import pallas as pl
from jax.experimental.pallas import tpu as pltpu
from jax.experimental.pallas import tpu_sc as plsc

N = 10000
D = 256
G = 64
NC = 2
NS = 16
NW = NC * NS
L = 16
ROWS_W = 320
NPAD = NW * ROWS_W
CHUNK = 64
NCHUNK = ROWS_W // CHUNK
SROWS = 80
ZROWS = SROWS // NS
TAIL_W = NW - 1
TAIL_ROWS = N - TAIL_W * ROWS_W


def _sc_body(x_hbm, bidx_hbm, xsum_hbm, cnt_hbm,
             xbuf, idxbuf, ones, zx, zc, ssum, scnt):
    c = lax.axis_index("c")
    s = lax.axis_index("s")
    w = c * NS + s

    zvec = jnp.zeros((L,), jnp.float32)
    for r in range(ZROWS):
        for k in range(D // L):
            zx[r, pl.ds(k * L, L)] = zvec
        zc[r] = zvec
    onev = jnp.ones((L,), jnp.float32)
    for r in range(CHUNK):
        ones[r] = onev

    pltpu.sync_copy(zx, ssum.at[pl.ds(s * ZROWS, ZROWS)])
    pltpu.sync_copy(zc, scnt.at[pl.ds(s * ZROWS, ZROWS)])

    pltpu.sync_copy(bidx_hbm.at[w], idxbuf)

    @pl.when(w < TAIL_W)
    def _():
        pltpu.sync_copy(x_hbm.at[pl.ds(w * ROWS_W, ROWS_W)], xbuf)

    @pl.when(w == TAIL_W)
    def _():
        pltpu.sync_copy(x_hbm.at[pl.ds(w * ROWS_W, TAIL_ROWS)],
                        xbuf.at[pl.ds(0, TAIL_ROWS)])

    plsc.subcore_barrier()

    for j in range(NCHUNK):
        pltpu.sync_copy(xbuf.at[pl.ds(j * CHUNK, CHUNK)],
                        ssum.at[idxbuf.at[j]], add=True)
        pltpu.sync_copy(ones, scnt.at[idxbuf.at[j]], add=True)

    plsc.subcore_barrier()

    gpt = G // NS
    pltpu.sync_copy(ssum.at[pl.ds(s * gpt, gpt)],
                    xsum_hbm.at[c].at[pl.ds(s * gpt, gpt)])
    pltpu.sync_copy(scnt.at[pl.ds(s * gpt, gpt)],
                    cnt_hbm.at[c].at[pl.ds(s * gpt, gpt)])


@jax.jit
def _sc_segment(x, bidx):
    mesh = plsc.VectorSubcoreMesh(core_axis_name="c", subcore_axis_name="s",
                                  num_cores=NC, num_subcores=NS)
    return pl.kernel(
        _sc_body,
        out_type=(jax.ShapeDtypeStruct((NC, G, D), jnp.float32),
                  jax.ShapeDtypeStruct((NC, G, L), jnp.float32)),
        mesh=mesh,
        compiler_params=pltpu.CompilerParams(use_tc_tiling_on_sc=False),
        scratch_types=[
            pltpu.VMEM((ROWS_W, D), jnp.float32),
            pltpu.VMEM((NCHUNK, CHUNK), jnp.int32),
            pltpu.VMEM((CHUNK, L), jnp.float32),
            pltpu.VMEM((ZROWS, D), jnp.float32),
            pltpu.VMEM((ZROWS, L), jnp.float32),
            pltpu.VMEM_SHARED((SROWS, D), jnp.float32),
            pltpu.VMEM_SHARED((SROWS, L), jnp.float32),
        ],
    )(x, bidx)


def _tc_combine_body(xsum_ref, cnt_ref, u_ref, out_ref):
    ssum = xsum_ref[0] + xsum_ref[1]
    cnt = cnt_ref[0] + cnt_ref[1]
    tot = jnp.sum(ssum, axis=1)
    counts = jnp.sum(cnt, axis=1) * (1.0 / L)
    usum = jnp.sum(u_ref[...], axis=1)
    out_ref[...] = usum + tot / jnp.maximum(counts, 1.0)


@jax.jit
def _tc_combine(xsum, cnt, u):
    return pl.pallas_call(
        _tc_combine_body,
        out_shape=jax.ShapeDtypeStruct((G,), jnp.float32),
    )(xsum, cnt, u)


def kernel(x, edge_index, edge_attr, u, batch):
    del edge_index, edge_attr
    b = batch.astype(jnp.int32)
    bpad = jnp.concatenate(
        [b, jnp.full((NPAD - N,), G, jnp.int32)]).reshape(NW, NCHUNK, CHUNK)
    xsum, cnt = _sc_segment(x, bpad)
    return _tc_combine(xsum, cnt, u)
